# baseline (speedup 1.0000x reference)
import jax
import jax.numpy as jnp
from jax import lax
from jax.experimental import pallas as pl
from jax.experimental.pallas import tpu as pltpu, tpu_sc as plsc

_N_ITEMS = 1000000
_HIST = 200


_CHUNK = 31264
_LAST_BASE = _N_ITEMS - _CHUNK
_FILL_ITERS = 244
_GROUPS = (_HIST + 15) // 16


def _sc_body(titem, trating, out_tr, tidx_v, tval_v, chunk, sem_in, sem_out):
    wid = lax.axis_index("s") * 2 + lax.axis_index("c")
    base = pl.multiple_of(
        jnp.where(wid == 31, _LAST_BASE, wid * _CHUNK).astype(jnp.int32), 8)
    lane = lax.iota(jnp.int32, 16)

    cps = [pltpu.async_copy(titem, tidx_v.at[pl.ds(0, _HIST)], sem_in),
           pltpu.async_copy(trating, tval_v.at[pl.ds(0, _HIST)], sem_in)]

    nvec = jnp.full((16,), jnp.nan, jnp.float32)

    def fill(i, carry):
        off = i * 128
        for u in range(8):
            chunk[pl.ds(off + u * 16, 16)] = nvec
        return carry
    lax.fori_loop(0, _FILL_ITERS, fill, 0)
    chunk[pl.ds(_FILL_ITERS * 128, 16)] = nvec
    chunk[pl.ds(_FILL_ITERS * 128 + 16, 16)] = nvec

    for cp in cps:
        cp.wait()

    for g in range(_GROUPS):
        loc = tidx_v[pl.ds(g * 16, 16)] - base
        val = tval_v[pl.ds(g * 16, 16)]
        owned = (loc >= 0) & (loc < _CHUNK)
        rem = _HIST - g * 16
        if rem < 16:
            owned = owned & (lane < rem)

        @pl.when(jnp.any(owned))
        def _():
            for j in range(16):
                plsc.store_scatter(chunk, [loc], val,
                                   mask=owned & (lane == j))

    pltpu.async_copy(chunk, out_tr.at[pl.ds(base, _CHUNK)], sem_out).wait()


def _sc_leaf(target_item, target_rating):
    mesh = plsc.VectorSubcoreMesh(core_axis_name="c", subcore_axis_name="s")
    call = pl.kernel(
        _sc_body,
        out_type=jax.ShapeDtypeStruct((_N_ITEMS,), jnp.float32),
        mesh=mesh,
        scratch_types=(
            pltpu.VMEM((_HIST + 16,), jnp.int32),
            pltpu.VMEM((_HIST + 16,), jnp.float32),
            pltpu.VMEM((_CHUNK,), jnp.float32),
            pltpu.SemaphoreType.DMA,
            pltpu.SemaphoreType.DMA,
        ),
        compiler_params=pltpu.CompilerParams(needs_layout_passes=False),
        name="flat_input_sc",
    )
    return call(target_item, target_rating)


_TC_BLOCK = 262144
_TC_GRID = 4


def _tc_body(user_s, item_s, rating_s, tuser_s, out_block, urep_ref, turep_ref):
    i = pl.program_id(0)
    base = i * _TC_BLOCK
    out_block[...] = jnp.zeros((_TC_BLOCK,), jnp.float32)

    lane128 = lax.iota(jnp.int32, 128)

    def scat(k, carry):
        idx = item_s[k] - base
        ok = jnp.logical_and(idx >= 0, idx < _TC_BLOCK)

        @pl.when(ok)
        def _():
            seg = pl.multiple_of((idx // 128) * 128, 128)
            off = idx % 128
            row = out_block[pl.ds(seg, 128)]
            out_block[pl.ds(seg, 128)] = jnp.where(
                lane128 == off, rating_s[k], row)
        return carry
    lax.fori_loop(0, _HIST, scat, 0)

    @pl.when(i == 0)
    def _():
        urep_ref[...] = jnp.full((_HIST,), user_s[0], jnp.int32)
        turep_ref[...] = jnp.full((_HIST,), tuser_s[0], jnp.int32)


def _tc_leaf(user, item, rating, target_user):
    return pl.pallas_call(
        _tc_body,
        grid=(_TC_GRID,),
        in_specs=[
            pl.BlockSpec(memory_space=pltpu.SMEM),
            pl.BlockSpec(memory_space=pltpu.SMEM),
            pl.BlockSpec(memory_space=pltpu.SMEM),
            pl.BlockSpec(memory_space=pltpu.SMEM),
        ],
        out_specs=[
            pl.BlockSpec((_TC_BLOCK,), lambda i: (i,)),
            pl.BlockSpec((_HIST,), lambda i: (0,)),
            pl.BlockSpec((_HIST,), lambda i: (0,)),
        ],
        out_shape=[
            jax.ShapeDtypeStruct((_N_ITEMS,), jnp.float32),
            jax.ShapeDtypeStruct((_HIST,), jnp.int32),
            jax.ShapeDtypeStruct((_HIST,), jnp.int32),
        ],
        name="flat_input_tc",
    )(user, item, rating, target_user)


def kernel(user, item, rating, target_user, target_item, target_rating):
    full_tr = _sc_leaf(target_item, target_rating)
    full_r, urep, turep = _tc_leaf(user, item, rating, target_user)
    return (urep, full_r, turep, full_tr)


# speedup vs baseline: 1.2759x; 1.2759x over previous
import jax
import jax.numpy as jnp
from jax import lax
from jax.experimental import pallas as pl
from jax.experimental.pallas import tpu as pltpu, tpu_sc as plsc

_N_ITEMS = 1000000
_HIST = 200


_CHUNK = 31264
_LAST_BASE = _N_ITEMS - _CHUNK
_FILL_ITERS = 244
_GROUPS = (_HIST + 15) // 16


def _sc_body(titem, trating, out_tr, tidx_v, tval_v, chunk, sem_in, sem_out):
    wid = lax.axis_index("s") * 2 + lax.axis_index("c")
    base = pl.multiple_of(
        jnp.where(wid == 31, _LAST_BASE, wid * _CHUNK).astype(jnp.int32), 8)
    lane = lax.iota(jnp.int32, 16)

    cps = [pltpu.async_copy(titem, tidx_v.at[pl.ds(0, _HIST)], sem_in),
           pltpu.async_copy(trating, tval_v.at[pl.ds(0, _HIST)], sem_in)]

    nvec = jnp.full((16,), jnp.nan, jnp.float32)

    def fill(i, carry):
        off = i * 128
        for u in range(8):
            chunk[pl.ds(off + u * 16, 16)] = nvec
        return carry
    lax.fori_loop(0, _FILL_ITERS, fill, 0)
    chunk[pl.ds(_FILL_ITERS * 128, 16)] = nvec
    chunk[pl.ds(_FILL_ITERS * 128 + 16, 16)] = nvec

    for cp in cps:
        cp.wait()

    for g in range(_GROUPS):
        loc = tidx_v[pl.ds(g * 16, 16)] - base
        val = tval_v[pl.ds(g * 16, 16)]
        owned = (loc >= 0) & (loc < _CHUNK)
        rem = _HIST - g * 16
        if rem < 16:
            owned = owned & (lane < rem)

        @pl.when(jnp.any(owned))
        def _():
            for j in range(16):
                plsc.store_scatter(chunk, [loc], val,
                                   mask=owned & (lane == j))

    pltpu.async_copy(chunk, out_tr.at[pl.ds(base, _CHUNK)], sem_out).wait()


def _sc_leaf(target_item, target_rating):
    mesh = plsc.VectorSubcoreMesh(core_axis_name="c", subcore_axis_name="s")
    call = pl.kernel(
        _sc_body,
        out_type=jax.ShapeDtypeStruct((_N_ITEMS,), jnp.float32),
        mesh=mesh,
        scratch_types=(
            pltpu.VMEM((_HIST + 16,), jnp.int32),
            pltpu.VMEM((_HIST + 16,), jnp.float32),
            pltpu.VMEM((_CHUNK,), jnp.float32),
            pltpu.SemaphoreType.DMA,
            pltpu.SemaphoreType.DMA,
        ),
        compiler_params=pltpu.CompilerParams(needs_layout_passes=False),
        name="flat_input_sc",
    )
    return call(target_item, target_rating)


def _tc_body(user_s, item_s, rating_s, tuser_s, out_ref, urep_ref, turep_ref):
    out_ref[...] = jnp.zeros((_N_ITEMS,), jnp.float32)

    lane128 = lax.iota(jnp.int32, 128)
    for k in range(_HIST):
        idx = item_s[k]
        seg = pl.multiple_of((idx // 128) * 128, 128)
        off = idx % 128
        row = out_ref[pl.ds(seg, 128)]
        out_ref[pl.ds(seg, 128)] = jnp.where(lane128 == off, rating_s[k], row)

    urep_ref[...] = jnp.full((_HIST,), user_s[0], jnp.int32)
    turep_ref[...] = jnp.full((_HIST,), tuser_s[0], jnp.int32)


def _tc_leaf(user, item, rating, target_user):
    return pl.pallas_call(
        _tc_body,
        in_specs=[
            pl.BlockSpec(memory_space=pltpu.SMEM),
            pl.BlockSpec(memory_space=pltpu.SMEM),
            pl.BlockSpec(memory_space=pltpu.SMEM),
            pl.BlockSpec(memory_space=pltpu.SMEM),
        ],
        out_shape=[
            jax.ShapeDtypeStruct((_N_ITEMS,), jnp.float32),
            jax.ShapeDtypeStruct((_HIST,), jnp.int32),
            jax.ShapeDtypeStruct((_HIST,), jnp.int32),
        ],
        name="flat_input_tc",
    )(user, item, rating, target_user)


def kernel(user, item, rating, target_user, target_item, target_rating):
    full_tr = _sc_leaf(target_item, target_rating)
    full_r, urep, turep = _tc_leaf(user, item, rating, target_user)
    return (urep, full_r, turep, full_tr)


# speedup vs baseline: 1.3042x; 1.0222x over previous
import jax
import jax.numpy as jnp
from jax import lax
from jax.experimental import pallas as pl
from jax.experimental.pallas import tpu as pltpu, tpu_sc as plsc

_N_ITEMS = 1000000
_HIST = 200


_NW = 16
_CHUNK = 62528
_LAST_BASE = _N_ITEMS - _CHUNK
_FILL_ITERS = 488
_GROUPS = (_HIST + 15) // 16


def _sc_body(titem, trating, out_tr, tidx_v, tval_v, chunk, sem_in, sem_out):
    wid = lax.axis_index("s")
    base = pl.multiple_of(
        jnp.where(wid == _NW - 1, _LAST_BASE, wid * _CHUNK).astype(jnp.int32), 8)
    lane = lax.iota(jnp.int32, 16)

    cps = [pltpu.async_copy(titem, tidx_v.at[pl.ds(0, _HIST)], sem_in),
           pltpu.async_copy(trating, tval_v.at[pl.ds(0, _HIST)], sem_in)]

    nvec = jnp.full((16,), jnp.nan, jnp.float32)

    def fill(i, carry):
        off = i * 128
        for u in range(8):
            chunk[pl.ds(off + u * 16, 16)] = nvec
        return carry
    lax.fori_loop(0, _FILL_ITERS, fill, 0)
    for t in range((_CHUNK - _FILL_ITERS * 128) // 16):
        chunk[pl.ds(_FILL_ITERS * 128 + t * 16, 16)] = nvec

    for cp in cps:
        cp.wait()

    for g in range(_GROUPS):
        loc = tidx_v[pl.ds(g * 16, 16)] - base
        val = tval_v[pl.ds(g * 16, 16)]
        owned = (loc >= 0) & (loc < _CHUNK)
        rem = _HIST - g * 16
        if rem < 16:
            owned = owned & (lane < rem)

        @pl.when(jnp.any(owned))
        def _():
            for j in range(16):
                plsc.store_scatter(chunk, [loc], val,
                                   mask=owned & (lane == j))

    pltpu.async_copy(chunk, out_tr.at[pl.ds(base, _CHUNK)], sem_out).wait()


def _sc_leaf(target_item, target_rating):
    mesh = plsc.VectorSubcoreMesh(core_axis_name="c", subcore_axis_name="s",
                                  num_cores=1)
    call = pl.kernel(
        _sc_body,
        out_type=jax.ShapeDtypeStruct((_N_ITEMS,), jnp.float32),
        mesh=mesh,
        scratch_types=(
            pltpu.VMEM((_HIST + 16,), jnp.int32),
            pltpu.VMEM((_HIST + 16,), jnp.float32),
            pltpu.VMEM((_CHUNK,), jnp.float32),
            pltpu.SemaphoreType.DMA,
            pltpu.SemaphoreType.DMA,
        ),
        compiler_params=pltpu.CompilerParams(needs_layout_passes=False),
        name="flat_input_sc",
    )
    return call(target_item, target_rating)


def _tc_body(user_s, item_s, rating_s, tuser_s, out_ref, urep_ref, turep_ref):
    out_ref[...] = jnp.zeros((_N_ITEMS,), jnp.float32)

    lane128 = lax.iota(jnp.int32, 128)
    for k in range(_HIST):
        idx = item_s[k]
        seg = pl.multiple_of((idx // 128) * 128, 128)
        off = idx % 128
        row = out_ref[pl.ds(seg, 128)]
        out_ref[pl.ds(seg, 128)] = jnp.where(lane128 == off, rating_s[k], row)

    urep_ref[...] = jnp.full((_HIST,), user_s[0], jnp.int32)
    turep_ref[...] = jnp.full((_HIST,), tuser_s[0], jnp.int32)


def _tc_leaf(user, item, rating, target_user):
    return pl.pallas_call(
        _tc_body,
        in_specs=[
            pl.BlockSpec(memory_space=pltpu.SMEM),
            pl.BlockSpec(memory_space=pltpu.SMEM),
            pl.BlockSpec(memory_space=pltpu.SMEM),
            pl.BlockSpec(memory_space=pltpu.SMEM),
        ],
        out_shape=[
            jax.ShapeDtypeStruct((_N_ITEMS,), jnp.float32),
            jax.ShapeDtypeStruct((_HIST,), jnp.int32),
            jax.ShapeDtypeStruct((_HIST,), jnp.int32),
        ],
        name="flat_input_tc",
    )(user, item, rating, target_user)


def kernel(user, item, rating, target_user, target_item, target_rating):
    full_tr = _sc_leaf(target_item, target_rating)
    full_r, urep, turep = _tc_leaf(user, item, rating, target_user)
    return (urep, full_r, turep, full_tr)
